# Initial kernel scaffold; baseline (speedup 1.0000x reference)
import jax
import jax.numpy as jnp
from jax.experimental import pallas as pl


def kernel(x, edge_index, batch, conv1_w1, conv1_b1, conv1_w2, conv1_b2, conv1_w3, conv1_b3, conv2_w1, conv2_b1, conv2_w2, conv2_b2, conv2_w3, conv2_b3, conv3_w1, conv3_b1, conv3_w2, conv3_b2, conv3_w3, conv3_b3, gn1_weight, gn1_bias, gn1_alpha, gn2_weight, gn2_bias, gn2_alpha, dense_w, dense_b, out_w, out_b):
    raise NotImplementedError("write your pallas kernel here")


# speedup vs baseline: 1.1380x; 1.1380x over previous
import functools

import jax
import jax.numpy as jnp
from jax import lax
from jax.experimental import pallas as pl
from jax.experimental.pallas import tpu as pltpu


def _node1_body(x_ref, wd_ref, wb_ref, p_ref, q_ref):
    x = x_ref[...]
    p_ref[...] = jnp.dot(x, wd_ref[...], preferred_element_type=jnp.float32)
    q_ref[...] = jnp.dot(x, wb_ref[...], preferred_element_type=jnp.float32)


def _node_body(s_ref, cnt_ref, gnw_ref, gnb_ref, gna_ref, wd_ref, wb_ref,
               p_ref, q_ref):
    s = s_ref[...]
    h = (s[0] + s[1]) / jnp.maximum(cnt_ref[...], 1.0)
    mean = jnp.mean(h, axis=0, keepdims=True)
    out = h - gna_ref[...] * mean
    var = jnp.mean(out * out, axis=0, keepdims=True)
    h = gnw_ref[...] * out * jax.lax.rsqrt(var + 1e-5) + gnb_ref[...]
    p_ref[...] = jnp.dot(h, wd_ref[...], preferred_element_type=jnp.float32)
    q_ref[...] = jnp.dot(h, wb_ref[...], preferred_element_type=jnp.float32)


def _edge_mlp_body(pre_ref, b1_ref, w2_ref, b2_ref, w3_ref, b3_ref, z_ref):
    h = jnp.maximum(pre_ref[...] + b1_ref[...], 0.0)
    h = jnp.dot(h, w2_ref[...], preferred_element_type=jnp.float32) + b2_ref[...]
    h = jnp.maximum(h, 0.0)
    z_ref[...] = jnp.dot(h, w3_ref[...], preferred_element_type=jnp.float32) + b3_ref[...]


def _head_body(s_ref, cnt_ref, batch_ref, dw_ref, db_ref, ow_ref, ob_ref,
               out_ref):
    s = s_ref[...]
    h = (s[0] + s[1]) / jnp.maximum(cnt_ref[...], 1.0)
    n, _ = h.shape
    g = out_ref.shape[0]
    gids = jax.lax.broadcasted_iota(jnp.int32, (n, g), 1)
    oh = (batch_ref[...] == gids).astype(jnp.float32)
    dn = (((0,), (0,)), ((), ()))
    pooled_s = jax.lax.dot_general(oh, h, dn, preferred_element_type=jnp.float32)
    cnt_g = jax.lax.dot_general(oh, jnp.ones((n, 1), jnp.float32), dn,
                                preferred_element_type=jnp.float32)
    pooled = pooled_s / jnp.maximum(cnt_g, 1.0)
    d = jnp.maximum(jnp.dot(pooled, dw_ref[...],
                            preferred_element_type=jnp.float32) + db_ref[...], 0.0)
    logits = jnp.dot(d, ow_ref[...],
                     preferred_element_type=jnp.float32) + ob_ref[...]
    m = jnp.max(logits, axis=1, keepdims=True)
    e = jnp.exp(logits - m)
    out_ref[...] = e / jnp.sum(e, axis=1, keepdims=True)


def _vmem_specs(k):
    return [pl.BlockSpec(memory_space=pltpu.ANY if False else pltpu.VMEM)
            for _ in range(k)]


def _node1(x, wd, wb):
    n = x.shape[0]
    h = wd.shape[1]
    return pl.pallas_call(
        _node1_body,
        out_shape=(jax.ShapeDtypeStruct((n, h), jnp.float32),
                   jax.ShapeDtypeStruct((n, h), jnp.float32)),
    )(x, wd, wb)


def _node(s, cnt, gnw, gnb, gna, wd, wb):
    n = s.shape[1]
    h = wd.shape[1]
    return pl.pallas_call(
        _node_body,
        out_shape=(jax.ShapeDtypeStruct((n, h), jnp.float32),
                   jax.ShapeDtypeStruct((n, h), jnp.float32)),
    )(s, cnt, gnw, gnb, gna, wd, wb)


def _edge_mlp(pre, b1, w2, b2, w3, b3, block_rows=2560):
    e, h = pre.shape
    assert e % block_rows == 0
    grid = e // block_rows
    return pl.pallas_call(
        _edge_mlp_body,
        grid=(grid,),
        in_specs=[
            pl.BlockSpec((block_rows, h), lambda i: (i, 0)),
            pl.BlockSpec((1, h), lambda i: (0, 0)),
            pl.BlockSpec((h, h), lambda i: (0, 0)),
            pl.BlockSpec((1, h), lambda i: (0, 0)),
            pl.BlockSpec((h, h), lambda i: (0, 0)),
            pl.BlockSpec((1, h), lambda i: (0, 0)),
        ],
        out_specs=pl.BlockSpec((block_rows, h), lambda i: (i, 0)),
        out_shape=jax.ShapeDtypeStruct((e, h), jnp.float32),
    )(pre, b1, w2, b2, w3, b3)


def _head(s, cnt, batch2d, dw, db, ow, ob, g):
    c = ow.shape[1]
    return pl.pallas_call(
        _head_body,
        out_shape=jax.ShapeDtypeStruct((g, c), jnp.float32),
    )(s, cnt, batch2d, dw, db, ow, ob)


def kernel(x, edge_index, batch,
           conv1_w1, conv1_b1, conv1_w2, conv1_b2, conv1_w3, conv1_b3,
           conv2_w1, conv2_b1, conv2_w2, conv2_b2, conv2_w3, conv2_b3,
           conv3_w1, conv3_b1, conv3_w2, conv3_b2, conv3_w3, conv3_b3,
           gn1_weight, gn1_bias, gn1_alpha, gn2_weight, gn2_bias, gn2_alpha,
           dense_w, dense_b, out_w, out_b):
    n, f_in = x.shape
    e = edge_index.shape[1]
    g = 128
    src = edge_index[0]
    dst = edge_index[1]

    def split_w(w):
        fi = w.shape[0] // 2
        return w[:fi] - w[fi:], w[fi:]

    w1d, w1b = split_w(conv1_w1)
    w2d, w2b = split_w(conv2_w1)
    w3d, w3b = split_w(conv3_w1)
    r = lambda v: v.reshape(1, -1)

    cnt = jax.ops.segment_sum(jnp.ones((e,), jnp.float32), dst,
                              num_segments=n).reshape(n, 1)

    def gather(p, q):
        return jnp.take(p, dst, axis=0) + jnp.take(q, src, axis=0)

    def scatter(z):
        s = jax.ops.segment_sum(z, dst, num_segments=n)
        return jnp.stack([s, jnp.zeros_like(s)], axis=0)

    p, q = _node1(x, w1d, w1b)
    z = _edge_mlp(gather(p, q), r(conv1_b1), conv1_w2, r(conv1_b2),
                  conv1_w3, r(conv1_b3))
    s = scatter(z)

    p, q = _node(s, cnt, r(gn1_weight), r(gn1_bias), r(gn1_alpha), w2d, w2b)
    z = _edge_mlp(gather(p, q), r(conv2_b1), conv2_w2, r(conv2_b2),
                  conv2_w3, r(conv2_b3))
    s = scatter(z)

    p, q = _node(s, cnt, r(gn2_weight), r(gn2_bias), r(gn2_alpha), w3d, w3b)
    z = _edge_mlp(gather(p, q), r(conv3_b1), conv3_w2, r(conv3_b2),
                  conv3_w3, r(conv3_b3))
    s = scatter(z)

    return _head(s, cnt, batch.reshape(n, 1), dense_w, r(dense_b),
                 out_w, r(out_b), g)


# speedup vs baseline: 2.0324x; 1.7859x over previous
import functools

import jax
import jax.numpy as jnp
from jax import lax
from jax.experimental import pallas as pl
from jax.experimental.pallas import tpu as pltpu
from jax.experimental.pallas import tpu_sc as plsc

_NC = 2
_NS = 16
_NW = _NC * _NS
_EB = 128


def _sc_gather(p, q, dst, src):
    n, h = p.shape
    e = dst.shape[0]
    nb = e // _EB
    mesh = plsc.VectorSubcoreMesh(core_axis_name="c", subcore_axis_name="s")

    @functools.partial(
        pl.kernel,
        out_type=jax.ShapeDtypeStruct((e, h), jnp.float32),
        mesh=mesh,
        scratch_types=[
            pltpu.VMEM((_EB,), jnp.int32),
            pltpu.VMEM((_EB,), jnp.int32),
            pltpu.VMEM((_EB, h), jnp.float32),
            pltpu.SemaphoreType.DMA,
        ],
        compiler_params=pltpu.CompilerParams(use_tc_tiling_on_sc=False),
    )
    def body(p_hbm, q_hbm, dst_hbm, src_hbm, out_hbm, idx_d, idx_s, rows, sem):
        wid = lax.axis_index("c") * _NS + lax.axis_index("s")
        nb_w = nb // _NW + jnp.where(wid < nb % _NW, 1, 0)

        def step(i, _):
            base = (wid + i * _NW) * _EB
            pltpu.sync_copy(dst_hbm.at[pl.ds(base, _EB)], idx_d)
            pltpu.sync_copy(src_hbm.at[pl.ds(base, _EB)], idx_s)
            pltpu.async_copy(p_hbm.at[idx_d], rows, sem).wait()
            pltpu.async_copy(q_hbm.at[idx_s], rows, sem, add=True).wait()
            pltpu.sync_copy(rows, out_hbm.at[pl.ds(base, _EB)])
            return 0

        lax.fori_loop(0, nb_w, step, 0)

    return body(p, q, dst, src)


def _node1_body(x_ref, wd_ref, wb_ref, p_ref, q_ref):
    x = x_ref[...]
    p_ref[...] = jnp.dot(x, wd_ref[...], preferred_element_type=jnp.float32)
    q_ref[...] = jnp.dot(x, wb_ref[...], preferred_element_type=jnp.float32)


def _node_body(s_ref, cnt_ref, gnw_ref, gnb_ref, gna_ref, wd_ref, wb_ref,
               p_ref, q_ref):
    s = s_ref[...]
    h = (s[0] + s[1]) / jnp.maximum(cnt_ref[...], 1.0)
    mean = jnp.mean(h, axis=0, keepdims=True)
    out = h - gna_ref[...] * mean
    var = jnp.mean(out * out, axis=0, keepdims=True)
    h = gnw_ref[...] * out * jax.lax.rsqrt(var + 1e-5) + gnb_ref[...]
    p_ref[...] = jnp.dot(h, wd_ref[...], preferred_element_type=jnp.float32)
    q_ref[...] = jnp.dot(h, wb_ref[...], preferred_element_type=jnp.float32)


def _edge_mlp_body(pre_ref, b1_ref, w2_ref, b2_ref, w3_ref, b3_ref, z_ref):
    h = jnp.maximum(pre_ref[...] + b1_ref[...], 0.0)
    h = jnp.dot(h, w2_ref[...], preferred_element_type=jnp.float32) + b2_ref[...]
    h = jnp.maximum(h, 0.0)
    z_ref[...] = jnp.dot(h, w3_ref[...], preferred_element_type=jnp.float32) + b3_ref[...]


def _head_body(s_ref, cnt_ref, batch_ref, dw_ref, db_ref, ow_ref, ob_ref,
               out_ref):
    s = s_ref[...]
    h = (s[0] + s[1]) / jnp.maximum(cnt_ref[...], 1.0)
    n, _ = h.shape
    g = out_ref.shape[0]
    gids = jax.lax.broadcasted_iota(jnp.int32, (n, g), 1)
    oh = (batch_ref[...] == gids).astype(jnp.float32)
    dn = (((0,), (0,)), ((), ()))
    pooled_s = jax.lax.dot_general(oh, h, dn, preferred_element_type=jnp.float32)
    cnt_g = jax.lax.dot_general(oh, jnp.ones((n, 1), jnp.float32), dn,
                                preferred_element_type=jnp.float32)
    pooled = pooled_s / jnp.maximum(cnt_g, 1.0)
    d = jnp.maximum(jnp.dot(pooled, dw_ref[...],
                            preferred_element_type=jnp.float32) + db_ref[...], 0.0)
    logits = jnp.dot(d, ow_ref[...],
                     preferred_element_type=jnp.float32) + ob_ref[...]
    m = jnp.max(logits, axis=1, keepdims=True)
    e = jnp.exp(logits - m)
    out_ref[...] = e / jnp.sum(e, axis=1, keepdims=True)


def _vmem_specs(k):
    return [pl.BlockSpec(memory_space=pltpu.ANY if False else pltpu.VMEM)
            for _ in range(k)]


def _node1(x, wd, wb):
    n = x.shape[0]
    h = wd.shape[1]
    return pl.pallas_call(
        _node1_body,
        out_shape=(jax.ShapeDtypeStruct((n, h), jnp.float32),
                   jax.ShapeDtypeStruct((n, h), jnp.float32)),
    )(x, wd, wb)


def _node(s, cnt, gnw, gnb, gna, wd, wb):
    n = s.shape[1]
    h = wd.shape[1]
    return pl.pallas_call(
        _node_body,
        out_shape=(jax.ShapeDtypeStruct((n, h), jnp.float32),
                   jax.ShapeDtypeStruct((n, h), jnp.float32)),
    )(s, cnt, gnw, gnb, gna, wd, wb)


def _edge_mlp(pre, b1, w2, b2, w3, b3, block_rows=2560):
    e, h = pre.shape
    assert e % block_rows == 0
    grid = e // block_rows
    return pl.pallas_call(
        _edge_mlp_body,
        grid=(grid,),
        in_specs=[
            pl.BlockSpec((block_rows, h), lambda i: (i, 0)),
            pl.BlockSpec((1, h), lambda i: (0, 0)),
            pl.BlockSpec((h, h), lambda i: (0, 0)),
            pl.BlockSpec((1, h), lambda i: (0, 0)),
            pl.BlockSpec((h, h), lambda i: (0, 0)),
            pl.BlockSpec((1, h), lambda i: (0, 0)),
        ],
        out_specs=pl.BlockSpec((block_rows, h), lambda i: (i, 0)),
        out_shape=jax.ShapeDtypeStruct((e, h), jnp.float32),
    )(pre, b1, w2, b2, w3, b3)


def _head(s, cnt, batch2d, dw, db, ow, ob, g):
    c = ow.shape[1]
    return pl.pallas_call(
        _head_body,
        out_shape=jax.ShapeDtypeStruct((g, c), jnp.float32),
    )(s, cnt, batch2d, dw, db, ow, ob)


def kernel(x, edge_index, batch,
           conv1_w1, conv1_b1, conv1_w2, conv1_b2, conv1_w3, conv1_b3,
           conv2_w1, conv2_b1, conv2_w2, conv2_b2, conv2_w3, conv2_b3,
           conv3_w1, conv3_b1, conv3_w2, conv3_b2, conv3_w3, conv3_b3,
           gn1_weight, gn1_bias, gn1_alpha, gn2_weight, gn2_bias, gn2_alpha,
           dense_w, dense_b, out_w, out_b):
    n, f_in = x.shape
    e = edge_index.shape[1]
    g = 128
    src = edge_index[0]
    dst = edge_index[1]

    def split_w(w):
        fi = w.shape[0] // 2
        return w[:fi] - w[fi:], w[fi:]

    w1d, w1b = split_w(conv1_w1)
    w2d, w2b = split_w(conv2_w1)
    w3d, w3b = split_w(conv3_w1)
    r = lambda v: v.reshape(1, -1)

    cnt = jax.ops.segment_sum(jnp.ones((e,), jnp.float32), dst,
                              num_segments=n).reshape(n, 1)

    def gather(p, q):
        return _sc_gather(p, q, dst, src)

    def scatter(z):
        s = jax.ops.segment_sum(z, dst, num_segments=n)
        return jnp.stack([s, jnp.zeros_like(s)], axis=0)

    p, q = _node1(x, w1d, w1b)
    z = _edge_mlp(gather(p, q), r(conv1_b1), conv1_w2, r(conv1_b2),
                  conv1_w3, r(conv1_b3))
    s = scatter(z)

    p, q = _node(s, cnt, r(gn1_weight), r(gn1_bias), r(gn1_alpha), w2d, w2b)
    z = _edge_mlp(gather(p, q), r(conv2_b1), conv2_w2, r(conv2_b2),
                  conv2_w3, r(conv2_b3))
    s = scatter(z)

    p, q = _node(s, cnt, r(gn2_weight), r(gn2_bias), r(gn2_alpha), w3d, w3b)
    z = _edge_mlp(gather(p, q), r(conv3_b1), conv3_w2, r(conv3_b2),
                  conv3_w3, r(conv3_b3))
    s = scatter(z)

    return _head(s, cnt, batch.reshape(n, 1), dense_w, r(dense_b),
                 out_w, r(out_b), g)


# speedup vs baseline: 3.0333x; 1.4925x over previous
import functools

import jax
import jax.numpy as jnp
from jax import lax
from jax.experimental import pallas as pl
from jax.experimental.pallas import tpu as pltpu
from jax.experimental.pallas import tpu_sc as plsc

_NC = 2
_NS = 16
_NW = _NC * _NS
_EB = 128


def _sc_gather(p, q, dst, src):
    n, h = p.shape
    e = dst.shape[0]
    nb = e // _EB
    mesh = plsc.VectorSubcoreMesh(core_axis_name="c", subcore_axis_name="s")

    @functools.partial(
        pl.kernel,
        out_type=jax.ShapeDtypeStruct((e, h), jnp.float32),
        mesh=mesh,
        scratch_types=[
            pltpu.VMEM((_EB,), jnp.int32),
            pltpu.VMEM((_EB,), jnp.int32),
            pltpu.VMEM((_EB, h), jnp.float32),
            pltpu.SemaphoreType.DMA,
        ],
        compiler_params=pltpu.CompilerParams(use_tc_tiling_on_sc=False),
    )
    def body(p_hbm, q_hbm, dst_hbm, src_hbm, out_hbm, idx_d, idx_s, rows, sem):
        wid = lax.axis_index("c") * _NS + lax.axis_index("s")
        nb_w = nb // _NW + jnp.where(wid < nb % _NW, 1, 0)

        def step(i, _):
            base = (wid + i * _NW) * _EB
            pltpu.sync_copy(dst_hbm.at[pl.ds(base, _EB)], idx_d)
            pltpu.sync_copy(src_hbm.at[pl.ds(base, _EB)], idx_s)
            pltpu.async_copy(p_hbm.at[idx_d], rows, sem).wait()
            pltpu.async_copy(q_hbm.at[idx_s], rows, sem, add=True).wait()
            pltpu.sync_copy(rows, out_hbm.at[pl.ds(base, _EB)])
            return 0

        lax.fori_loop(0, nb_w, step, 0)

    return body(p, q, dst, src)


def _sc_scatter(z, dst, zeros_nh, ones_b, with_counts):
    e, h = z.shape
    n = zeros_nh.shape[0]
    nb = e // _EB
    rpt = n // _NS
    mesh = plsc.VectorSubcoreMesh(core_axis_name="c", subcore_axis_name="s")
    out_type = [jax.ShapeDtypeStruct((_NC, n, h), jnp.float32)]
    scratch = [
        pltpu.VMEM((_EB,), jnp.int32),
        pltpu.VMEM((_EB, h), jnp.float32),
        pltpu.VMEM((rpt, h), jnp.float32),
        pltpu.VMEM_SHARED((n, h), jnp.float32),
    ]
    hc = 8
    if with_counts:
        out_type.append(jax.ShapeDtypeStruct((_NC, n, hc), jnp.float32))
        scratch += [
            pltpu.VMEM((_EB, hc), jnp.float32),
            pltpu.VMEM_SHARED((n, hc), jnp.float32),
            pltpu.VMEM((rpt, hc), jnp.float32),
        ]

    def body(z_hbm, dst_hbm, zer_hbm, one_hbm, *rest):
        if with_counts:
            (s_out, c_out, idx_d, rows, zbuf, acc, ones_v, cacc, czbuf) = rest
        else:
            (s_out, idx_d, rows, zbuf, acc) = rest
        c = lax.axis_index("c")
        s = lax.axis_index("s")
        wid = c * _NS + s
        nb_w = nb // _NW + jnp.where(wid < nb % _NW, 1, 0)
        sl = pl.ds(s * rpt, rpt)

        pltpu.sync_copy(zer_hbm.at[sl], zbuf)
        pltpu.sync_copy(zbuf, acc.at[sl])
        if with_counts:
            pltpu.sync_copy(zer_hbm.at[sl, pl.ds(0, hc)], czbuf)
            pltpu.sync_copy(czbuf, cacc.at[sl])
            pltpu.sync_copy(one_hbm, ones_v)
        plsc.subcore_barrier()

        def step(i, _):
            base = (wid + i * _NW) * _EB
            pltpu.sync_copy(dst_hbm.at[pl.ds(base, _EB)], idx_d)
            pltpu.sync_copy(z_hbm.at[pl.ds(base, _EB)], rows)
            pltpu.sync_copy(rows, acc.at[idx_d], add=True)
            if with_counts:
                pltpu.sync_copy(ones_v, cacc.at[idx_d], add=True)
            return 0

        lax.fori_loop(0, nb_w, step, 0)
        plsc.subcore_barrier()

        pltpu.sync_copy(acc.at[sl], zbuf)
        pltpu.sync_copy(zbuf, s_out.at[c, sl])
        if with_counts:
            pltpu.sync_copy(cacc.at[sl], czbuf)
            pltpu.sync_copy(czbuf, c_out.at[c, sl])

    fn = pl.kernel(
        body, out_type=tuple(out_type), mesh=mesh, scratch_types=scratch,
        compiler_params=pltpu.CompilerParams(use_tc_tiling_on_sc=False),
    )
    return fn(z, dst, zeros_nh, ones_b)


def _node1_body(x_ref, wd_ref, wb_ref, p_ref, q_ref):
    x = x_ref[...]
    p_ref[...] = jnp.dot(x, wd_ref[...], preferred_element_type=jnp.float32)
    q_ref[...] = jnp.dot(x, wb_ref[...], preferred_element_type=jnp.float32)


def _node_body(s_ref, cnt_ref, gnw_ref, gnb_ref, gna_ref, wd_ref, wb_ref,
               p_ref, q_ref):
    s = s_ref[...]
    h = (s[0] + s[1]) / jnp.maximum(cnt_ref[...], 1.0)
    mean = jnp.mean(h, axis=0, keepdims=True)
    out = h - gna_ref[...] * mean
    var = jnp.mean(out * out, axis=0, keepdims=True)
    h = gnw_ref[...] * out * jax.lax.rsqrt(var + 1e-5) + gnb_ref[...]
    p_ref[...] = jnp.dot(h, wd_ref[...], preferred_element_type=jnp.float32)
    q_ref[...] = jnp.dot(h, wb_ref[...], preferred_element_type=jnp.float32)


def _edge_mlp_body(pre_ref, b1_ref, w2_ref, b2_ref, w3_ref, b3_ref, z_ref):
    h = jnp.maximum(pre_ref[...] + b1_ref[...], 0.0)
    h = jnp.dot(h, w2_ref[...], preferred_element_type=jnp.float32) + b2_ref[...]
    h = jnp.maximum(h, 0.0)
    z_ref[...] = jnp.dot(h, w3_ref[...], preferred_element_type=jnp.float32) + b3_ref[...]


def _head_body(s_ref, cnt_ref, batch_ref, dw_ref, db_ref, ow_ref, ob_ref,
               out_ref):
    s = s_ref[...]
    h = (s[0] + s[1]) / jnp.maximum(cnt_ref[...], 1.0)
    n, _ = h.shape
    g = out_ref.shape[0]
    gids = jax.lax.broadcasted_iota(jnp.int32, (n, g), 1)
    oh = (batch_ref[...] == gids).astype(jnp.float32)
    dn = (((0,), (0,)), ((), ()))
    pooled_s = jax.lax.dot_general(oh, h, dn, preferred_element_type=jnp.float32)
    cnt_g = jax.lax.dot_general(oh, jnp.ones((n, 1), jnp.float32), dn,
                                preferred_element_type=jnp.float32)
    pooled = pooled_s / jnp.maximum(cnt_g, 1.0)
    d = jnp.maximum(jnp.dot(pooled, dw_ref[...],
                            preferred_element_type=jnp.float32) + db_ref[...], 0.0)
    logits = jnp.dot(d, ow_ref[...],
                     preferred_element_type=jnp.float32) + ob_ref[...]
    m = jnp.max(logits, axis=1, keepdims=True)
    e = jnp.exp(logits - m)
    out_ref[...] = e / jnp.sum(e, axis=1, keepdims=True)


def _vmem_specs(k):
    return [pl.BlockSpec(memory_space=pltpu.ANY if False else pltpu.VMEM)
            for _ in range(k)]


def _node1(x, wd, wb):
    n = x.shape[0]
    h = wd.shape[1]
    return pl.pallas_call(
        _node1_body,
        out_shape=(jax.ShapeDtypeStruct((n, h), jnp.float32),
                   jax.ShapeDtypeStruct((n, h), jnp.float32)),
    )(x, wd, wb)


def _node(s, cnt, gnw, gnb, gna, wd, wb):
    n = s.shape[1]
    h = wd.shape[1]
    return pl.pallas_call(
        _node_body,
        out_shape=(jax.ShapeDtypeStruct((n, h), jnp.float32),
                   jax.ShapeDtypeStruct((n, h), jnp.float32)),
    )(s, cnt, gnw, gnb, gna, wd, wb)


def _edge_mlp(pre, b1, w2, b2, w3, b3, block_rows=2560):
    e, h = pre.shape
    assert e % block_rows == 0
    grid = e // block_rows
    return pl.pallas_call(
        _edge_mlp_body,
        grid=(grid,),
        in_specs=[
            pl.BlockSpec((block_rows, h), lambda i: (i, 0)),
            pl.BlockSpec((1, h), lambda i: (0, 0)),
            pl.BlockSpec((h, h), lambda i: (0, 0)),
            pl.BlockSpec((1, h), lambda i: (0, 0)),
            pl.BlockSpec((h, h), lambda i: (0, 0)),
            pl.BlockSpec((1, h), lambda i: (0, 0)),
        ],
        out_specs=pl.BlockSpec((block_rows, h), lambda i: (i, 0)),
        out_shape=jax.ShapeDtypeStruct((e, h), jnp.float32),
    )(pre, b1, w2, b2, w3, b3)


def _head(s, cnt, batch2d, dw, db, ow, ob, g):
    c = ow.shape[1]
    return pl.pallas_call(
        _head_body,
        out_shape=jax.ShapeDtypeStruct((g, c), jnp.float32),
    )(s, cnt, batch2d, dw, db, ow, ob)


def kernel(x, edge_index, batch,
           conv1_w1, conv1_b1, conv1_w2, conv1_b2, conv1_w3, conv1_b3,
           conv2_w1, conv2_b1, conv2_w2, conv2_b2, conv2_w3, conv2_b3,
           conv3_w1, conv3_b1, conv3_w2, conv3_b2, conv3_w3, conv3_b3,
           gn1_weight, gn1_bias, gn1_alpha, gn2_weight, gn2_bias, gn2_alpha,
           dense_w, dense_b, out_w, out_b):
    n, f_in = x.shape
    e = edge_index.shape[1]
    g = 128
    src = edge_index[0]
    dst = edge_index[1]

    def split_w(w):
        fi = w.shape[0] // 2
        return w[:fi] - w[fi:], w[fi:]

    w1d, w1b = split_w(conv1_w1)
    w2d, w2b = split_w(conv2_w1)
    w3d, w3b = split_w(conv3_w1)
    r = lambda v: v.reshape(1, -1)

    h = 64
    zeros_nh = jnp.zeros((n, h), jnp.float32)
    ones_b = jnp.ones((_EB, 8), jnp.float32)

    def gather(p, q):
        return _sc_gather(p, q, dst, src)

    def scatter(z):
        (s,) = _sc_scatter(z, dst, zeros_nh, ones_b, with_counts=False)
        return s

    p, q = _node1(x, w1d, w1b)
    z = _edge_mlp(gather(p, q), r(conv1_b1), conv1_w2, r(conv1_b2),
                  conv1_w3, r(conv1_b3))
    s, cpart = _sc_scatter(z, dst, zeros_nh, ones_b, with_counts=True)
    cnt = cpart[0, :, :1] + cpart[1, :, :1]

    p, q = _node(s, cnt, r(gn1_weight), r(gn1_bias), r(gn1_alpha), w2d, w2b)
    z = _edge_mlp(gather(p, q), r(conv2_b1), conv2_w2, r(conv2_b2),
                  conv2_w3, r(conv2_b3))
    s = scatter(z)

    p, q = _node(s, cnt, r(gn2_weight), r(gn2_bias), r(gn2_alpha), w3d, w3b)
    z = _edge_mlp(gather(p, q), r(conv3_b1), conv3_w2, r(conv3_b2),
                  conv3_w3, r(conv3_b3))
    s = scatter(z)

    return _head(s, cnt, batch.reshape(n, 1), dense_w, r(dense_b),
                 out_w, r(out_b), g)


# speedup vs baseline: 4.1283x; 1.3610x over previous
import functools

import jax
import jax.numpy as jnp
from jax import lax
from jax.experimental import pallas as pl
from jax.experimental.pallas import tpu as pltpu
from jax.experimental.pallas import tpu_sc as plsc

_NC = 2
_NS = 16
_NW = _NC * _NS
_EB = 128


_CB = 10


def _sc_gather(p, q, dst2, src2):
    n, h = p.shape
    nb = dst2.shape[0]
    e = nb * _EB
    nch = nb // _CB
    ce = _CB * _EB
    mesh = plsc.VectorSubcoreMesh(core_axis_name="c", subcore_axis_name="s")

    @functools.partial(
        pl.kernel,
        out_type=jax.ShapeDtypeStruct((e, h), jnp.float32),
        mesh=mesh,
        scratch_types=[
            pltpu.VMEM((_CB, _EB), jnp.int32),
            pltpu.VMEM((_CB, _EB), jnp.int32),
            pltpu.VMEM((ce, h), jnp.float32),
            pltpu.SemaphoreType.DMA,
            pltpu.SemaphoreType.DMA,
        ],
        compiler_params=pltpu.CompilerParams(use_tc_tiling_on_sc=False),
    )
    def body(p_hbm, q_hbm, dst_hbm, src_hbm, out_hbm, idx_d, idx_s, rows,
             sem_i, sem_g):
        wid = lax.axis_index("c") * _NS + lax.axis_index("s")
        nch_w = nch // _NW + jnp.where(wid < nch % _NW, 1, 0)

        def step(i, _):
            ch = wid + i * _NW
            bb = ch * _CB
            eb = ch * ce
            a = pltpu.async_copy(dst_hbm.at[pl.ds(bb, _CB)], idx_d, sem_i)
            b = pltpu.async_copy(src_hbm.at[pl.ds(bb, _CB)], idx_s, sem_i)
            a.wait()
            b.wait()
            ds = [pltpu.async_copy(p_hbm.at[idx_d.at[j]],
                                   rows.at[pl.ds(j * _EB, _EB)], sem_g)
                  for j in range(_CB)]
            for d in ds:
                d.wait()
            ds = [pltpu.async_copy(q_hbm.at[idx_s.at[j]],
                                   rows.at[pl.ds(j * _EB, _EB)], sem_g,
                                   add=True)
                  for j in range(_CB)]
            for d in ds:
                d.wait()
            pltpu.sync_copy(rows, out_hbm.at[pl.ds(eb, ce)])
            return 0

        lax.fori_loop(0, nch_w, step, 0)

    return body(p, q, dst2, src2)


def _sc_scatter(z, dst2, zeros_nh, ones_b, with_counts):
    cb = 5
    e, h = z.shape
    n = zeros_nh.shape[0]
    nb = dst2.shape[0]
    nch = nb // cb
    ce = cb * _EB
    rpt = n // _NS
    wpt = 125
    nw = rpt // wpt
    mesh = plsc.VectorSubcoreMesh(core_axis_name="c", subcore_axis_name="s")
    out_type = [jax.ShapeDtypeStruct((_NC, n, h), jnp.float32)]
    hc = 8
    scratch = [
        pltpu.VMEM((cb, _EB), jnp.int32),
        pltpu.VMEM((ce, h), jnp.float32),
        pltpu.VMEM((wpt, h), jnp.float32),
        pltpu.VMEM_SHARED((n, h), jnp.float32),
        pltpu.SemaphoreType.DMA,
        pltpu.SemaphoreType.DMA,
    ]
    if with_counts:
        out_type.append(jax.ShapeDtypeStruct((_NC, n, hc), jnp.float32))
        scratch += [
            pltpu.VMEM((_EB, hc), jnp.float32),
            pltpu.VMEM_SHARED((n, hc), jnp.float32),
            pltpu.VMEM((wpt, hc), jnp.float32),
        ]

    def body(z_hbm, dst_hbm, zer_hbm, one_hbm, *rest):
        if with_counts:
            (s_out, c_out, idx_d, rows, zbuf, acc, sem_l, sem_s,
             ones_v, cacc, czbuf) = rest
        else:
            (s_out, idx_d, rows, zbuf, acc, sem_l, sem_s) = rest
        c = lax.axis_index("c")
        s = lax.axis_index("s")
        wid = c * _NS + s
        nch_w = nch // _NW + jnp.where(wid < nch % _NW, 1, 0)

        for k in range(nw):
            sl = pl.ds(s * rpt + k * wpt, wpt)
            pltpu.sync_copy(zer_hbm.at[sl], zbuf)
            pltpu.sync_copy(zbuf, acc.at[sl])
            if with_counts:
                pltpu.sync_copy(zer_hbm.at[sl, pl.ds(0, hc)], czbuf)
                pltpu.sync_copy(czbuf, cacc.at[sl])
        if with_counts:
            pltpu.sync_copy(one_hbm, ones_v)
        plsc.subcore_barrier()

        def step(i, _):
            ch = wid + i * _NW
            a = pltpu.async_copy(dst_hbm.at[pl.ds(ch * cb, cb)], idx_d,
                                 sem_l)
            b = pltpu.async_copy(z_hbm.at[pl.ds(ch * ce, ce)], rows, sem_l)
            a.wait()
            b.wait()
            ds = [pltpu.async_copy(rows.at[pl.ds(j * _EB, _EB)],
                                   acc.at[idx_d.at[j]], sem_s, add=True)
                  for j in range(cb)]
            if with_counts:
                ds += [pltpu.async_copy(ones_v, cacc.at[idx_d.at[j]], sem_s,
                                        add=True)
                       for j in range(cb)]
            for d in ds:
                d.wait()
            return 0

        lax.fori_loop(0, nch_w, step, 0)
        plsc.subcore_barrier()

        for k in range(nw):
            sl = pl.ds(s * rpt + k * wpt, wpt)
            pltpu.sync_copy(acc.at[sl], zbuf)
            pltpu.sync_copy(zbuf, s_out.at[c, sl])
            if with_counts:
                pltpu.sync_copy(cacc.at[sl], czbuf)
                pltpu.sync_copy(czbuf, c_out.at[c, sl])

    fn = pl.kernel(
        body, out_type=tuple(out_type), mesh=mesh, scratch_types=scratch,
        compiler_params=pltpu.CompilerParams(use_tc_tiling_on_sc=False),
    )
    return fn(z, dst2, zeros_nh, ones_b)


def _node1_body(x_ref, wd_ref, wb_ref, p_ref, q_ref):
    x = x_ref[...]
    p_ref[...] = jnp.dot(x, wd_ref[...], preferred_element_type=jnp.float32)
    q_ref[...] = jnp.dot(x, wb_ref[...], preferred_element_type=jnp.float32)


def _node_body(s_ref, cnt_ref, gnw_ref, gnb_ref, gna_ref, wd_ref, wb_ref,
               p_ref, q_ref):
    s = s_ref[...]
    h = (s[0] + s[1]) / jnp.maximum(cnt_ref[...], 1.0)
    mean = jnp.mean(h, axis=0, keepdims=True)
    out = h - gna_ref[...] * mean
    var = jnp.mean(out * out, axis=0, keepdims=True)
    h = gnw_ref[...] * out * jax.lax.rsqrt(var + 1e-5) + gnb_ref[...]
    p_ref[...] = jnp.dot(h, wd_ref[...], preferred_element_type=jnp.float32)
    q_ref[...] = jnp.dot(h, wb_ref[...], preferred_element_type=jnp.float32)


def _edge_mlp_body(pre_ref, b1_ref, w2_ref, b2_ref, w3_ref, b3_ref, z_ref):
    h = jnp.maximum(pre_ref[...] + b1_ref[...], 0.0)
    h = jnp.dot(h, w2_ref[...], preferred_element_type=jnp.float32) + b2_ref[...]
    h = jnp.maximum(h, 0.0)
    z_ref[...] = jnp.dot(h, w3_ref[...], preferred_element_type=jnp.float32) + b3_ref[...]


def _head_body(s_ref, cnt_ref, batch_ref, dw_ref, db_ref, ow_ref, ob_ref,
               out_ref):
    s = s_ref[...]
    h = (s[0] + s[1]) / jnp.maximum(cnt_ref[...], 1.0)
    n, _ = h.shape
    g = out_ref.shape[0]
    gids = jax.lax.broadcasted_iota(jnp.int32, (n, g), 1)
    oh = (batch_ref[...] == gids).astype(jnp.float32)
    dn = (((0,), (0,)), ((), ()))
    pooled_s = jax.lax.dot_general(oh, h, dn, preferred_element_type=jnp.float32)
    cnt_g = jax.lax.dot_general(oh, jnp.ones((n, 1), jnp.float32), dn,
                                preferred_element_type=jnp.float32)
    pooled = pooled_s / jnp.maximum(cnt_g, 1.0)
    d = jnp.maximum(jnp.dot(pooled, dw_ref[...],
                            preferred_element_type=jnp.float32) + db_ref[...], 0.0)
    logits = jnp.dot(d, ow_ref[...],
                     preferred_element_type=jnp.float32) + ob_ref[...]
    m = jnp.max(logits, axis=1, keepdims=True)
    e = jnp.exp(logits - m)
    out_ref[...] = e / jnp.sum(e, axis=1, keepdims=True)


def _vmem_specs(k):
    return [pl.BlockSpec(memory_space=pltpu.ANY if False else pltpu.VMEM)
            for _ in range(k)]


def _node1(x, wd, wb):
    n = x.shape[0]
    h = wd.shape[1]
    return pl.pallas_call(
        _node1_body,
        out_shape=(jax.ShapeDtypeStruct((n, h), jnp.float32),
                   jax.ShapeDtypeStruct((n, h), jnp.float32)),
    )(x, wd, wb)


def _node(s, cnt, gnw, gnb, gna, wd, wb):
    n = s.shape[1]
    h = wd.shape[1]
    return pl.pallas_call(
        _node_body,
        out_shape=(jax.ShapeDtypeStruct((n, h), jnp.float32),
                   jax.ShapeDtypeStruct((n, h), jnp.float32)),
    )(s, cnt, gnw, gnb, gna, wd, wb)


def _edge_mlp(pre, b1, w2, b2, w3, b3, block_rows=2560):
    e, h = pre.shape
    assert e % block_rows == 0
    grid = e // block_rows
    return pl.pallas_call(
        _edge_mlp_body,
        grid=(grid,),
        in_specs=[
            pl.BlockSpec((block_rows, h), lambda i: (i, 0)),
            pl.BlockSpec((1, h), lambda i: (0, 0)),
            pl.BlockSpec((h, h), lambda i: (0, 0)),
            pl.BlockSpec((1, h), lambda i: (0, 0)),
            pl.BlockSpec((h, h), lambda i: (0, 0)),
            pl.BlockSpec((1, h), lambda i: (0, 0)),
        ],
        out_specs=pl.BlockSpec((block_rows, h), lambda i: (i, 0)),
        out_shape=jax.ShapeDtypeStruct((e, h), jnp.float32),
    )(pre, b1, w2, b2, w3, b3)


def _head(s, cnt, batch2d, dw, db, ow, ob, g):
    c = ow.shape[1]
    return pl.pallas_call(
        _head_body,
        out_shape=jax.ShapeDtypeStruct((g, c), jnp.float32),
    )(s, cnt, batch2d, dw, db, ow, ob)


def kernel(x, edge_index, batch,
           conv1_w1, conv1_b1, conv1_w2, conv1_b2, conv1_w3, conv1_b3,
           conv2_w1, conv2_b1, conv2_w2, conv2_b2, conv2_w3, conv2_b3,
           conv3_w1, conv3_b1, conv3_w2, conv3_b2, conv3_w3, conv3_b3,
           gn1_weight, gn1_bias, gn1_alpha, gn2_weight, gn2_bias, gn2_alpha,
           dense_w, dense_b, out_w, out_b):
    n, f_in = x.shape
    e = edge_index.shape[1]
    g = 128
    src2 = edge_index[0].reshape(e // _EB, _EB)
    dst2 = edge_index[1].reshape(e // _EB, _EB)

    def split_w(w):
        fi = w.shape[0] // 2
        return w[:fi] - w[fi:], w[fi:]

    w1d, w1b = split_w(conv1_w1)
    w2d, w2b = split_w(conv2_w1)
    w3d, w3b = split_w(conv3_w1)
    r = lambda v: v.reshape(1, -1)

    h = 64
    zeros_nh = jnp.zeros((n, h), jnp.float32)
    ones_b = jnp.ones((_EB, 8), jnp.float32)

    def gather(p, q):
        return _sc_gather(p, q, dst2, src2)

    def scatter(z):
        (s,) = _sc_scatter(z, dst2, zeros_nh, ones_b, with_counts=False)
        return s

    p, q = _node1(x, w1d, w1b)
    z = _edge_mlp(gather(p, q), r(conv1_b1), conv1_w2, r(conv1_b2),
                  conv1_w3, r(conv1_b3))
    s, cpart = _sc_scatter(z, dst2, zeros_nh, ones_b, with_counts=True)
    cnt = cpart[0, :, :1] + cpart[1, :, :1]

    p, q = _node(s, cnt, r(gn1_weight), r(gn1_bias), r(gn1_alpha), w2d, w2b)
    z = _edge_mlp(gather(p, q), r(conv2_b1), conv2_w2, r(conv2_b2),
                  conv2_w3, r(conv2_b3))
    s = scatter(z)

    p, q = _node(s, cnt, r(gn2_weight), r(gn2_bias), r(gn2_alpha), w3d, w3b)
    z = _edge_mlp(gather(p, q), r(conv3_b1), conv3_w2, r(conv3_b2),
                  conv3_w3, r(conv3_b3))
    s = scatter(z)

    return _head(s, cnt, batch.reshape(n, 1), dense_w, r(dense_b),
                 out_w, r(out_b), g)


# speedup vs baseline: 8.7353x; 2.1160x over previous
import functools

import jax
import jax.numpy as jnp
from jax import lax
from jax.experimental import pallas as pl
from jax.experimental.pallas import tpu as pltpu
from jax.experimental.pallas import tpu_sc as plsc

_NC = 2
_NS = 16
_NW = _NC * _NS
_EB = 128


_CB = 10


def _sc_gather(p, q, dst2, src2):
    n, h = p.shape
    nb = dst2.shape[0]
    e = nb * _EB
    nch = nb // _CB
    ce = _CB * _EB
    mesh = plsc.VectorSubcoreMesh(core_axis_name="c", subcore_axis_name="s")

    @functools.partial(
        pl.kernel,
        out_type=jax.ShapeDtypeStruct((e, h), jnp.float32),
        mesh=mesh,
        scratch_types=[
            pltpu.VMEM((_CB, _EB), jnp.int32),
            pltpu.VMEM((_CB, _EB), jnp.int32),
            pltpu.VMEM((ce, h), jnp.float32),
            pltpu.SemaphoreType.DMA,
            pltpu.SemaphoreType.DMA,
        ],
        compiler_params=pltpu.CompilerParams(use_tc_tiling_on_sc=False, needs_layout_passes=False),
    )
    def body(p_hbm, q_hbm, dst_hbm, src_hbm, out_hbm, idx_d, idx_s, rows,
             sem_i, sem_g):
        wid = lax.axis_index("c") * _NS + lax.axis_index("s")
        nch_w = nch // _NW + jnp.where(wid < nch % _NW, 1, 0)

        def step(i, _):
            ch = wid + i * _NW
            bb = ch * _CB
            eb = ch * ce
            a = pltpu.async_copy(dst_hbm.at[pl.ds(bb, _CB)], idx_d, sem_i)
            b = pltpu.async_copy(src_hbm.at[pl.ds(bb, _CB)], idx_s, sem_i)
            a.wait()
            b.wait()
            ds = [pltpu.async_copy(p_hbm.at[idx_d.at[j]],
                                   rows.at[pl.ds(j * _EB, _EB)], sem_g)
                  for j in range(_CB)]
            for d in ds:
                d.wait()
            ds = [pltpu.async_copy(q_hbm.at[idx_s.at[j]],
                                   rows.at[pl.ds(j * _EB, _EB)], sem_g,
                                   add=True)
                  for j in range(_CB)]
            for d in ds:
                d.wait()
            pltpu.sync_copy(rows, out_hbm.at[pl.ds(eb, ce)])
            return 0

        lax.fori_loop(0, nch_w, step, 0)

    return body(p, q, dst2, src2)


def _sc_scatter(z, dst2, zeros_nh, zeros_n, with_counts):
    cb = 5
    e, h = z.shape
    n = zeros_nh.shape[0]
    nb = dst2.shape[0]
    nch = nb // cb
    ce = cb * _EB
    rpt = n // _NS
    wpt = 125
    nw = rpt // wpt
    mesh = plsc.VectorSubcoreMesh(core_axis_name="c", subcore_axis_name="s")
    out_type = [jax.ShapeDtypeStruct((_NC, n, h), jnp.float32)]
    scratch = [
        pltpu.VMEM((cb, _EB), jnp.int32),
        pltpu.VMEM((ce, h), jnp.float32),
        pltpu.VMEM((wpt, h), jnp.float32),
        pltpu.VMEM_SHARED((n, h), jnp.float32),
        pltpu.SemaphoreType.DMA,
        pltpu.SemaphoreType.DMA,
    ]
    if with_counts:
        out_type.append(jax.ShapeDtypeStruct((_NW, n), jnp.float32))
        scratch.append(pltpu.VMEM((n,), jnp.float32))

    def body(z_hbm, dst_hbm, zer_hbm, zer1_hbm, *rest):
        if with_counts:
            (s_out, c_out, idx_d, rows, zbuf, acc, sem_l, sem_s, cnt_v) = rest
        else:
            (s_out, idx_d, rows, zbuf, acc, sem_l, sem_s) = rest
        c = lax.axis_index("c")
        s = lax.axis_index("s")
        wid = c * _NS + s
        nch_w = nch // _NW + jnp.where(wid < nch % _NW, 1, 0)

        for k in range(nw):
            sl = pl.ds(s * rpt + k * wpt, wpt)
            pltpu.sync_copy(zer_hbm.at[sl], zbuf)
            pltpu.sync_copy(zbuf, acc.at[sl])
        if with_counts:
            pltpu.sync_copy(zer1_hbm, cnt_v)
        plsc.subcore_barrier()

        ones16 = jnp.ones((16,), jnp.float32)

        def step(i, _):
            ch = wid + i * _NW
            a = pltpu.async_copy(dst_hbm.at[pl.ds(ch * cb, cb)], idx_d,
                                 sem_l)
            b = pltpu.async_copy(z_hbm.at[pl.ds(ch * ce, ce)], rows, sem_l)
            a.wait()
            b.wait()
            ds = [pltpu.async_copy(rows.at[pl.ds(j * _EB, _EB)],
                                   acc.at[idx_d.at[j]], sem_s, add=True)
                  for j in range(cb)]
            if with_counts:
                for j in range(cb):
                    for k in range(_EB // 16):
                        iv = idx_d[j, pl.ds(k * 16, 16)]
                        plsc.addupdate_scatter(cnt_v, [iv], ones16)
            for d in ds:
                d.wait()
            return 0

        lax.fori_loop(0, nch_w, step, 0)
        plsc.subcore_barrier()

        for k in range(nw):
            sl = pl.ds(s * rpt + k * wpt, wpt)
            pltpu.sync_copy(acc.at[sl], zbuf)
            pltpu.sync_copy(zbuf, s_out.at[c, sl])
        if with_counts:
            pltpu.sync_copy(cnt_v, c_out.at[wid])

    fn = pl.kernel(
        body, out_type=tuple(out_type), mesh=mesh, scratch_types=scratch,
        compiler_params=pltpu.CompilerParams(use_tc_tiling_on_sc=False, needs_layout_passes=False),
    )
    return fn(z, dst2, zeros_nh, zeros_n)


def _node1_body(x_ref, wd_ref, wb_ref, p_ref, q_ref):
    x = x_ref[...]
    p_ref[...] = jnp.dot(x, wd_ref[...], preferred_element_type=jnp.float32)
    q_ref[...] = jnp.dot(x, wb_ref[...], preferred_element_type=jnp.float32)


def _node_body(s_ref, cp_ref, gnw_ref, gnb_ref, gna_ref, wd_ref, wb_ref,
               p_ref, q_ref):
    s = s_ref[...]
    dn = (((0,), (0,)), ((), ()))
    cnt = jax.lax.dot_general(cp_ref[...], jnp.ones((cp_ref.shape[0], 1),
                                                    jnp.float32), dn,
                              preferred_element_type=jnp.float32)
    h = (s[0] + s[1]) / jnp.maximum(cnt, 1.0)
    mean = jnp.mean(h, axis=0, keepdims=True)
    out = h - gna_ref[...] * mean
    var = jnp.mean(out * out, axis=0, keepdims=True)
    h = gnw_ref[...] * out * jax.lax.rsqrt(var + 1e-5) + gnb_ref[...]
    p_ref[...] = jnp.dot(h, wd_ref[...], preferred_element_type=jnp.float32)
    q_ref[...] = jnp.dot(h, wb_ref[...], preferred_element_type=jnp.float32)


def _edge_mlp_body(pre_ref, b1_ref, w2_ref, b2_ref, w3_ref, b3_ref, z_ref):
    h = jnp.maximum(pre_ref[...] + b1_ref[...], 0.0)
    h = jnp.dot(h, w2_ref[...], preferred_element_type=jnp.float32) + b2_ref[...]
    h = jnp.maximum(h, 0.0)
    z_ref[...] = jnp.dot(h, w3_ref[...], preferred_element_type=jnp.float32) + b3_ref[...]


def _head_body(s_ref, cp_ref, batch_ref, dw_ref, db_ref, ow_ref, ob_ref,
               out_ref):
    s = s_ref[...]
    dn = (((0,), (0,)), ((), ()))
    cnt = jax.lax.dot_general(cp_ref[...], jnp.ones((cp_ref.shape[0], 1),
                                                    jnp.float32), dn,
                              preferred_element_type=jnp.float32)
    h = (s[0] + s[1]) / jnp.maximum(cnt, 1.0)
    n, _ = h.shape
    g = out_ref.shape[0]
    gids = jax.lax.broadcasted_iota(jnp.int32, (n, g), 1)
    oh = (batch_ref[...] == gids).astype(jnp.float32)
    pooled_s = jax.lax.dot_general(oh, h, dn, preferred_element_type=jnp.float32)
    cnt_g = jax.lax.dot_general(oh, jnp.ones((n, 1), jnp.float32), dn,
                                preferred_element_type=jnp.float32)
    pooled = pooled_s / jnp.maximum(cnt_g, 1.0)
    d = jnp.maximum(jnp.dot(pooled, dw_ref[...],
                            preferred_element_type=jnp.float32) + db_ref[...], 0.0)
    logits = jnp.dot(d, ow_ref[...],
                     preferred_element_type=jnp.float32) + ob_ref[...]
    m = jnp.max(logits, axis=1, keepdims=True)
    e = jnp.exp(logits - m)
    out_ref[...] = e / jnp.sum(e, axis=1, keepdims=True)


def _node1(x, wd, wb):
    n = x.shape[0]
    h = wd.shape[1]
    return pl.pallas_call(
        _node1_body,
        out_shape=(jax.ShapeDtypeStruct((n, h), jnp.float32),
                   jax.ShapeDtypeStruct((n, h), jnp.float32)),
    )(x, wd, wb)


def _node(s, cpart, gnw, gnb, gna, wd, wb):
    n = s.shape[1]
    h = wd.shape[1]
    return pl.pallas_call(
        _node_body,
        out_shape=(jax.ShapeDtypeStruct((n, h), jnp.float32),
                   jax.ShapeDtypeStruct((n, h), jnp.float32)),
    )(s, cpart, gnw, gnb, gna, wd, wb)


def _edge_mlp(pre, b1, w2, b2, w3, b3, block_rows=3200):
    e2, hh = pre.shape
    assert e2 % block_rows == 0
    grid = e2 // block_rows
    return pl.pallas_call(
        _edge_mlp_body,
        grid=(grid,),
        in_specs=[
            pl.BlockSpec((block_rows, hh), lambda i: (i, 0)),
            pl.BlockSpec((1, hh), lambda i: (0, 0)),
            pl.BlockSpec((hh, hh), lambda i: (0, 0)),
            pl.BlockSpec((1, hh), lambda i: (0, 0)),
            pl.BlockSpec((hh, hh), lambda i: (0, 0)),
            pl.BlockSpec((1, hh), lambda i: (0, 0)),
        ],
        out_specs=pl.BlockSpec((block_rows, hh), lambda i: (i, 0)),
        out_shape=jax.ShapeDtypeStruct((e2, hh), jnp.float32),
    )(pre, b1, w2, b2, w3, b3)


def _head(s, cpart, batch2d, dw, db, ow, ob, g):
    c = ow.shape[1]
    return pl.pallas_call(
        _head_body,
        out_shape=jax.ShapeDtypeStruct((g, c), jnp.float32),
    )(s, cpart, batch2d, dw, db, ow, ob)


def kernel(x, edge_index, batch,
           conv1_w1, conv1_b1, conv1_w2, conv1_b2, conv1_w3, conv1_b3,
           conv2_w1, conv2_b1, conv2_w2, conv2_b2, conv2_w3, conv2_b3,
           conv3_w1, conv3_b1, conv3_w2, conv3_b2, conv3_w3, conv3_b3,
           gn1_weight, gn1_bias, gn1_alpha, gn2_weight, gn2_bias, gn2_alpha,
           dense_w, dense_b, out_w, out_b):
    n, f_in = x.shape
    e = edge_index.shape[1]
    g = 128
    src2 = edge_index[0].reshape(e // _EB, _EB)
    dst2 = edge_index[1].reshape(e // _EB, _EB)

    def split_w(w):
        fi = w.shape[0] // 2
        return w[:fi] - w[fi:], w[fi:]

    w1d, w1b = split_w(conv1_w1)
    w2d, w2b = split_w(conv2_w1)
    w3d, w3b = split_w(conv3_w1)
    r = lambda v: v.reshape(1, -1)
    rp = lambda v: jnp.concatenate([v, v]).reshape(1, -1)

    def blk(w):
        hh = w.shape[0]
        z = jnp.zeros((2 * hh, 2 * hh), jnp.float32)
        return z.at[:hh, :hh].set(w).at[hh:, hh:].set(w)

    h = 64
    zeros_nh = jnp.zeros((n, h), jnp.float32)
    zeros_n = jnp.zeros((n,), jnp.float32)

    def gather(p, q):
        return _sc_gather(p, q, dst2, src2).reshape(e // 2, 2 * h)

    def mlp(pre2, b1, w2, b2, w3, b3):
        z2 = _edge_mlp(pre2, rp(b1), blk(w2), rp(b2), blk(w3), rp(b3))
        return z2.reshape(e, h)

    def scatter(z):
        (s,) = _sc_scatter(z, dst2, zeros_nh, zeros_n, with_counts=False)
        return s

    p, q = _node1(x, w1d, w1b)
    z = mlp(gather(p, q), conv1_b1, conv1_w2, conv1_b2, conv1_w3, conv1_b3)
    s, cpart = _sc_scatter(z, dst2, zeros_nh, zeros_n, with_counts=True)

    p, q = _node(s, cpart, r(gn1_weight), r(gn1_bias), r(gn1_alpha), w2d, w2b)
    z = mlp(gather(p, q), conv2_b1, conv2_w2, conv2_b2, conv2_w3, conv2_b3)
    s = scatter(z)

    p, q = _node(s, cpart, r(gn2_weight), r(gn2_bias), r(gn2_alpha), w3d, w3b)
    z = mlp(gather(p, q), conv3_b1, conv3_w2, conv3_b2, conv3_w3, conv3_b3)
    s = scatter(z)

    return _head(s, cpart, batch.reshape(n, 1), dense_w, r(dense_b),
                 out_w, r(out_b), g)


# speedup vs baseline: 9.1369x; 1.0460x over previous
import functools

import jax
import jax.numpy as jnp
from jax import lax
from jax.experimental import pallas as pl
from jax.experimental.pallas import tpu as pltpu
from jax.experimental.pallas import tpu_sc as plsc

_NC = 2
_NS = 16
_NW = _NC * _NS
_EB = 128


_CB = 10


def _sc_gather(p, q, dst2, src2):
    n, h = p.shape
    nb = dst2.shape[0]
    e = nb * _EB
    nch = nb // _CB
    ce = _CB * _EB
    mesh = plsc.VectorSubcoreMesh(core_axis_name="c", subcore_axis_name="s")

    @functools.partial(
        pl.kernel,
        out_type=jax.ShapeDtypeStruct((e, h), jnp.float32),
        mesh=mesh,
        scratch_types=[
            pltpu.VMEM((_CB, _EB), jnp.int32),
            pltpu.VMEM((_CB, _EB), jnp.int32),
            pltpu.VMEM((ce, h), jnp.float32),
            pltpu.SemaphoreType.DMA,
            pltpu.SemaphoreType.DMA,
        ],
        compiler_params=pltpu.CompilerParams(use_tc_tiling_on_sc=False, needs_layout_passes=False),
    )
    def body(p_hbm, q_hbm, dst_hbm, src_hbm, out_hbm, idx_d, idx_s, rows,
             sem_i, sem_g):
        wid = lax.axis_index("c") * _NS + lax.axis_index("s")
        nch_w = nch // _NW + jnp.where(wid < nch % _NW, 1, 0)

        def step(i, _):
            ch = wid + i * _NW
            bb = ch * _CB
            eb = ch * ce
            a = pltpu.async_copy(dst_hbm.at[pl.ds(bb, _CB)], idx_d, sem_i)
            b = pltpu.async_copy(src_hbm.at[pl.ds(bb, _CB)], idx_s, sem_i)
            a.wait()
            b.wait()
            ds = [pltpu.async_copy(p_hbm.at[idx_d.at[j]],
                                   rows.at[pl.ds(j * _EB, _EB)], sem_g)
                  for j in range(_CB)]
            for d in ds:
                d.wait()
            ds = [pltpu.async_copy(q_hbm.at[idx_s.at[j]],
                                   rows.at[pl.ds(j * _EB, _EB)], sem_g,
                                   add=True)
                  for j in range(_CB)]
            for d in ds:
                d.wait()
            pltpu.sync_copy(rows, out_hbm.at[pl.ds(eb, ce)])
            return 0

        lax.fori_loop(0, nch_w, step, 0)

    return body(p, q, dst2, src2)


def _sc_scatter(z, dst2, zeros_nh, zeros_n, with_counts):
    cb = 5
    e, h = z.shape
    n = zeros_nh.shape[0]
    nb = dst2.shape[0]
    nch = nb // cb
    ce = cb * _EB
    rpt = n // _NS
    wpt = 125
    nw = rpt // wpt
    mesh = plsc.VectorSubcoreMesh(core_axis_name="c", subcore_axis_name="s")
    out_type = [jax.ShapeDtypeStruct((_NC, n, h), jnp.float32)]
    scratch = [
        pltpu.VMEM((cb, _EB), jnp.int32),
        pltpu.VMEM((ce, h), jnp.float32),
        pltpu.VMEM((wpt, h), jnp.float32),
        pltpu.VMEM_SHARED((n, h), jnp.float32),
        pltpu.SemaphoreType.DMA,
        pltpu.SemaphoreType.DMA,
    ]
    if with_counts:
        out_type.append(jax.ShapeDtypeStruct((_NW, n), jnp.float32))
        scratch.append(pltpu.VMEM((n,), jnp.float32))

    def body(z_hbm, dst_hbm, zer_hbm, zer1_hbm, *rest):
        if with_counts:
            (s_out, c_out, idx_d, rows, zbuf, acc, sem_l, sem_s, cnt_v) = rest
        else:
            (s_out, idx_d, rows, zbuf, acc, sem_l, sem_s) = rest
        c = lax.axis_index("c")
        s = lax.axis_index("s")
        wid = c * _NS + s
        nch_w = nch // _NW + jnp.where(wid < nch % _NW, 1, 0)

        for k in range(nw):
            sl = pl.ds(s * rpt + k * wpt, wpt)
            pltpu.sync_copy(zer_hbm.at[sl], zbuf)
            pltpu.sync_copy(zbuf, acc.at[sl])
        if with_counts:
            pltpu.sync_copy(zer1_hbm, cnt_v)
        plsc.subcore_barrier()

        ones16 = jnp.ones((16,), jnp.float32)

        def step(i, _):
            ch = wid + i * _NW
            a = pltpu.async_copy(dst_hbm.at[pl.ds(ch * cb, cb)], idx_d,
                                 sem_l)
            b = pltpu.async_copy(z_hbm.at[pl.ds(ch * ce, ce)], rows, sem_l)
            a.wait()
            b.wait()
            ds = [pltpu.async_copy(rows.at[pl.ds(j * _EB, _EB)],
                                   acc.at[idx_d.at[j]], sem_s, add=True)
                  for j in range(cb)]
            if with_counts:
                for j in range(cb):
                    for k in range(_EB // 16):
                        iv = idx_d[j, pl.ds(k * 16, 16)]
                        plsc.addupdate_scatter(cnt_v, [iv], ones16)
            for d in ds:
                d.wait()
            return 0

        lax.fori_loop(0, nch_w, step, 0)
        plsc.subcore_barrier()

        for k in range(nw):
            sl = pl.ds(s * rpt + k * wpt, wpt)
            pltpu.sync_copy(acc.at[sl], zbuf)
            pltpu.sync_copy(zbuf, s_out.at[c, sl])
        if with_counts:
            pltpu.sync_copy(cnt_v, c_out.at[wid])

    fn = pl.kernel(
        body, out_type=tuple(out_type), mesh=mesh, scratch_types=scratch,
        compiler_params=pltpu.CompilerParams(use_tc_tiling_on_sc=False, needs_layout_passes=False),
    )
    return fn(z, dst2, zeros_nh, zeros_n)


def _node1_body(x_ref, wd_ref, wb_ref, p_ref, q_ref):
    x = x_ref[...]
    p_ref[...] = jnp.dot(x, wd_ref[...], preferred_element_type=jnp.float32)
    q_ref[...] = jnp.dot(x, wb_ref[...], preferred_element_type=jnp.float32)


def _node_body(sa_ref, sb_ref, cpa_ref, cpb_ref, gnw_ref, gnb_ref,
               gna_ref, wd_ref, wb_ref, p_ref, q_ref):
    sa = sa_ref[...]
    sb = sb_ref[...]
    dn = (((0,), (0,)), ((), ()))
    one32 = jnp.ones((cpa_ref.shape[0], 1), jnp.float32)
    cnt = (jax.lax.dot_general(cpa_ref[...], one32, dn,
                               preferred_element_type=jnp.float32)
           + jax.lax.dot_general(cpb_ref[...], one32, dn,
                                 preferred_element_type=jnp.float32))
    h = (sa[0] + sa[1] + sb[0] + sb[1]) / jnp.maximum(cnt, 1.0)
    mean = jnp.mean(h, axis=0, keepdims=True)
    out = h - gna_ref[...] * mean
    var = jnp.mean(out * out, axis=0, keepdims=True)
    h = gnw_ref[...] * out * jax.lax.rsqrt(var + 1e-5) + gnb_ref[...]
    p_ref[...] = jnp.dot(h, wd_ref[...], preferred_element_type=jnp.float32)
    q_ref[...] = jnp.dot(h, wb_ref[...], preferred_element_type=jnp.float32)


def _edge_mlp_body(pre_ref, b1_ref, w2_ref, b2_ref, w3_ref, b3_ref, z_ref):
    h = jnp.maximum(pre_ref[...] + b1_ref[...], 0.0)
    h = jnp.dot(h, w2_ref[...], preferred_element_type=jnp.float32) + b2_ref[...]
    h = jnp.maximum(h, 0.0)
    z_ref[...] = jnp.dot(h, w3_ref[...], preferred_element_type=jnp.float32) + b3_ref[...]


def _head_body(sa_ref, sb_ref, cpa_ref, cpb_ref, batch_ref, dw_ref,
               db_ref, ow_ref, ob_ref, out_ref):
    sa = sa_ref[...]
    sb = sb_ref[...]
    dn = (((0,), (0,)), ((), ()))
    one32 = jnp.ones((cpa_ref.shape[0], 1), jnp.float32)
    cnt = (jax.lax.dot_general(cpa_ref[...], one32, dn,
                               preferred_element_type=jnp.float32)
           + jax.lax.dot_general(cpb_ref[...], one32, dn,
                                 preferred_element_type=jnp.float32))
    h = (sa[0] + sa[1] + sb[0] + sb[1]) / jnp.maximum(cnt, 1.0)
    n, _ = h.shape
    g = out_ref.shape[0]
    gids = jax.lax.broadcasted_iota(jnp.int32, (n, g), 1)
    oh = (batch_ref[...] == gids).astype(jnp.float32)
    pooled_s = jax.lax.dot_general(oh, h, dn, preferred_element_type=jnp.float32)
    cnt_g = jax.lax.dot_general(oh, jnp.ones((n, 1), jnp.float32), dn,
                                preferred_element_type=jnp.float32)
    pooled = pooled_s / jnp.maximum(cnt_g, 1.0)
    d = jnp.maximum(jnp.dot(pooled, dw_ref[...],
                            preferred_element_type=jnp.float32) + db_ref[...], 0.0)
    logits = jnp.dot(d, ow_ref[...],
                     preferred_element_type=jnp.float32) + ob_ref[...]
    m = jnp.max(logits, axis=1, keepdims=True)
    e = jnp.exp(logits - m)
    out_ref[...] = e / jnp.sum(e, axis=1, keepdims=True)


def _node1(x, wd, wb):
    n = x.shape[0]
    h = wd.shape[1]
    return pl.pallas_call(
        _node1_body,
        out_shape=(jax.ShapeDtypeStruct((n, h), jnp.float32),
                   jax.ShapeDtypeStruct((n, h), jnp.float32)),
    )(x, wd, wb)


def _node(sa, sb, cpa, cpb, gnw, gnb, gna, wd, wb):
    n = sa.shape[1]
    h = wd.shape[1]
    return pl.pallas_call(
        _node_body,
        out_shape=(jax.ShapeDtypeStruct((n, h), jnp.float32),
                   jax.ShapeDtypeStruct((n, h), jnp.float32)),
    )(sa, sb, cpa, cpb, gnw, gnb, gna, wd, wb)


def _edge_mlp(pre, b1, w2, b2, w3, b3, block_rows=3200):
    e2, hh = pre.shape
    assert e2 % block_rows == 0
    grid = e2 // block_rows
    return pl.pallas_call(
        _edge_mlp_body,
        grid=(grid,),
        in_specs=[
            pl.BlockSpec((block_rows, hh), lambda i: (i, 0)),
            pl.BlockSpec((1, hh), lambda i: (0, 0)),
            pl.BlockSpec((hh, hh), lambda i: (0, 0)),
            pl.BlockSpec((1, hh), lambda i: (0, 0)),
            pl.BlockSpec((hh, hh), lambda i: (0, 0)),
            pl.BlockSpec((1, hh), lambda i: (0, 0)),
        ],
        out_specs=pl.BlockSpec((block_rows, hh), lambda i: (i, 0)),
        out_shape=jax.ShapeDtypeStruct((e2, hh), jnp.float32),
    )(pre, b1, w2, b2, w3, b3)


def _head(sa, sb, cpa, cpb, batch2d, dw, db, ow, ob, g):
    c = ow.shape[1]
    return pl.pallas_call(
        _head_body,
        out_shape=jax.ShapeDtypeStruct((g, c), jnp.float32),
    )(sa, sb, cpa, cpb, batch2d, dw, db, ow, ob)


def kernel(x, edge_index, batch,
           conv1_w1, conv1_b1, conv1_w2, conv1_b2, conv1_w3, conv1_b3,
           conv2_w1, conv2_b1, conv2_w2, conv2_b2, conv2_w3, conv2_b3,
           conv3_w1, conv3_b1, conv3_w2, conv3_b2, conv3_w3, conv3_b3,
           gn1_weight, gn1_bias, gn1_alpha, gn2_weight, gn2_bias, gn2_alpha,
           dense_w, dense_b, out_w, out_b):
    n, f_in = x.shape
    e = edge_index.shape[1]
    g = 128
    src2 = edge_index[0].reshape(e // _EB, _EB)
    dst2 = edge_index[1].reshape(e // _EB, _EB)

    def split_w(w):
        fi = w.shape[0] // 2
        return w[:fi] - w[fi:], w[fi:]

    w1d, w1b = split_w(conv1_w1)
    w2d, w2b = split_w(conv2_w1)
    w3d, w3b = split_w(conv3_w1)
    r = lambda v: v.reshape(1, -1)
    rp = lambda v: jnp.concatenate([v, v]).reshape(1, -1)

    def blk(w):
        hh = w.shape[0]
        z = jnp.zeros((2 * hh, 2 * hh), jnp.float32)
        return z.at[:hh, :hh].set(w).at[hh:, hh:].set(w)

    h = 64
    zeros_nh = jnp.zeros((n, h), jnp.float32)
    zeros_n = jnp.zeros((n,), jnp.float32)
    nb = e // _EB
    halves = [(dst2[:nb // 2], src2[:nb // 2]),
              (dst2[nb // 2:], src2[nb // 2:])]

    def layer(p, q, b1, w2, b2, w3, b3, with_counts):
        zs = []
        for dh, sh in halves:
            pre2 = _sc_gather(p, q, dh, sh).reshape(-1, 2 * h)
            z2 = _edge_mlp(pre2, rp(b1), blk(w2), rp(b2), blk(w3), rp(b3))
            zs.append(z2.reshape(-1, h))
        outs = []
        for (dh, _), z in zip(halves, zs):
            outs.append(_sc_scatter(z, dh, zeros_nh, zeros_n,
                                    with_counts=with_counts))
        return outs

    p, q = _node1(x, w1d, w1b)
    (sa, cpa), (sb, cpb) = layer(p, q, conv1_b1, conv1_w2, conv1_b2,
                                 conv1_w3, conv1_b3, True)

    p, q = _node(sa, sb, cpa, cpb, r(gn1_weight), r(gn1_bias), r(gn1_alpha),
                 w2d, w2b)
    (sa2,), (sb2,) = layer(p, q, conv2_b1, conv2_w2, conv2_b2, conv2_w3,
                           conv2_b3, False)

    p, q = _node(sa2, sb2, cpa, cpb, r(gn2_weight), r(gn2_bias),
                 r(gn2_alpha), w3d, w3b)
    (sa3,), (sb3,) = layer(p, q, conv3_b1, conv3_w2, conv3_b2, conv3_w3,
                           conv3_b3, False)

    return _head(sa3, sb3, cpa, cpb, batch.reshape(n, 1), dense_w,
                 r(dense_b), out_w, r(out_b), g)


# speedup vs baseline: 9.3576x; 1.0242x over previous
import functools

import jax
import jax.numpy as jnp
from jax import lax
from jax.experimental import pallas as pl
from jax.experimental.pallas import tpu as pltpu
from jax.experimental.pallas import tpu_sc as plsc

_NC = 2
_NS = 16
_NW = _NC * _NS
_EB = 128


_CB = 10


def _sc_gather(p, q, dst2, src2):
    n, h = p.shape
    nb = dst2.shape[0]
    e = nb * _EB
    nch = nb // _CB
    ce = _CB * _EB
    mesh = plsc.VectorSubcoreMesh(core_axis_name="c", subcore_axis_name="s")

    @functools.partial(
        pl.kernel,
        out_type=jax.ShapeDtypeStruct((e, h), jnp.float32),
        mesh=mesh,
        scratch_types=[
            pltpu.VMEM((_CB, _EB), jnp.int32),
            pltpu.VMEM((_CB, _EB), jnp.int32),
            pltpu.VMEM((_CB, _EB), jnp.int32),
            pltpu.VMEM((_CB, _EB), jnp.int32),
            pltpu.VMEM((ce, h), jnp.float32),
            pltpu.SemaphoreType.DMA,
            pltpu.SemaphoreType.DMA,
            pltpu.SemaphoreType.DMA,
            pltpu.SemaphoreType.DMA,
        ],
        compiler_params=pltpu.CompilerParams(use_tc_tiling_on_sc=False,
                                             needs_layout_passes=False),
    )
    def body(p_hbm, q_hbm, dst_hbm, src_hbm, out_hbm, idx_d0, idx_s0,
             idx_d1, idx_s1, rows, sem_i0, sem_i1, sem_g, sem_w):
        wid = lax.axis_index("c") * _NS + lax.axis_index("s")
        nch_w = nch // _NW + jnp.where(wid < nch % _NW, 1, 0)
        bufs = ((idx_d0, idx_s0, sem_i0), (idx_d1, idx_s1, sem_i1))

        def fire_idx(i, bd, bs, sem):
            @pl.when(i < nch_w)
            def _():
                ch = wid + i * _NW
                pltpu.async_copy(dst_hbm.at[pl.ds(ch * _CB, _CB)], bd, sem)
                pltpu.async_copy(src_hbm.at[pl.ds(ch * _CB, _CB)], bs, sem)

        fire_idx(0, *bufs[0])
        fire_idx(1, *bufs[1])

        def chunk(i, bd, bs, sem):
            @pl.when(i < nch_w)
            def _():
                ch = wid + i * _NW

                @pl.when(i >= 1)
                def _():
                    pltpu.make_async_copy(rows, out_hbm.at[pl.ds(0, ce)],
                                          sem_w).wait()
                pltpu.make_async_copy(dst_hbm.at[pl.ds(0, _CB)], bd, sem).wait()
                pltpu.make_async_copy(src_hbm.at[pl.ds(0, _CB)], bs, sem).wait()
                dsp = [pltpu.async_copy(p_hbm.at[bd.at[j]],
                                        rows.at[pl.ds(j * _EB, _EB)], sem_g)
                       for j in range(_CB)]
                dsq = []
                for j in range(_CB):
                    dsp[j].wait()
                    dsq.append(pltpu.async_copy(q_hbm.at[bs.at[j]],
                                                rows.at[pl.ds(j * _EB, _EB)],
                                                sem_g, add=True))
                for d in dsq:
                    d.wait()
                fire_idx(i + 2, bd, bs, sem)
                pltpu.async_copy(rows, out_hbm.at[pl.ds(ch * ce, ce)], sem_w)

        def pair(t, _):
            chunk(2 * t, *bufs[0])
            chunk(2 * t + 1, *bufs[1])
            return 0

        lax.fori_loop(0, (nch_w + 1) // 2, pair, 0)
        pltpu.make_async_copy(rows, out_hbm.at[pl.ds(0, ce)], sem_w).wait()

    return body(p, q, dst2, src2)


def _sc_scatter(z, dst2, zeros_nh, zeros_n, with_counts):
    cb = 5
    e, h = z.shape
    n = zeros_nh.shape[0]
    nb = dst2.shape[0]
    nch = nb // cb
    ce = cb * _EB
    rpt = n // _NS
    wpt = 125
    nw = rpt // wpt
    mesh = plsc.VectorSubcoreMesh(core_axis_name="c", subcore_axis_name="s")
    out_type = [jax.ShapeDtypeStruct((_NC, n, h), jnp.float32)]
    scratch = [
        pltpu.VMEM((cb, _EB), jnp.int32),
        pltpu.VMEM((ce, h), jnp.float32),
        pltpu.VMEM((wpt, h), jnp.float32),
        pltpu.VMEM_SHARED((n, h), jnp.float32),
        pltpu.SemaphoreType.DMA,
        pltpu.SemaphoreType.DMA,
    ]
    if with_counts:
        out_type.append(jax.ShapeDtypeStruct((_NW, n), jnp.float32))
        scratch.append(pltpu.VMEM((n,), jnp.float32))

    def body(z_hbm, dst_hbm, zer_hbm, zer1_hbm, *rest):
        if with_counts:
            (s_out, c_out, idx_d, rows, zbuf, acc, sem_l, sem_s, cnt_v) = rest
        else:
            (s_out, idx_d, rows, zbuf, acc, sem_l, sem_s) = rest
        c = lax.axis_index("c")
        s = lax.axis_index("s")
        wid = c * _NS + s
        nch_w = nch // _NW + jnp.where(wid < nch % _NW, 1, 0)

        for k in range(nw):
            sl = pl.ds(s * rpt + k * wpt, wpt)
            pltpu.sync_copy(zer_hbm.at[sl], zbuf)
            pltpu.sync_copy(zbuf, acc.at[sl])
        if with_counts:
            pltpu.sync_copy(zer1_hbm, cnt_v)
        plsc.subcore_barrier()

        ones16 = jnp.ones((16,), jnp.float32)

        def step(i, _):
            ch = wid + i * _NW
            a = pltpu.async_copy(dst_hbm.at[pl.ds(ch * cb, cb)], idx_d,
                                 sem_l)
            b = pltpu.async_copy(z_hbm.at[pl.ds(ch * ce, ce)], rows, sem_l)
            a.wait()
            b.wait()
            ds = [pltpu.async_copy(rows.at[pl.ds(j * _EB, _EB)],
                                   acc.at[idx_d.at[j]], sem_s, add=True)
                  for j in range(cb)]
            if with_counts:
                for j in range(cb):
                    for k in range(_EB // 16):
                        iv = idx_d[j, pl.ds(k * 16, 16)]
                        plsc.addupdate_scatter(cnt_v, [iv], ones16)
            for d in ds:
                d.wait()
            return 0

        lax.fori_loop(0, nch_w, step, 0)
        plsc.subcore_barrier()

        for k in range(nw):
            sl = pl.ds(s * rpt + k * wpt, wpt)
            pltpu.sync_copy(acc.at[sl], zbuf)
            pltpu.sync_copy(zbuf, s_out.at[c, sl])
        if with_counts:
            pltpu.sync_copy(cnt_v, c_out.at[wid])

    fn = pl.kernel(
        body, out_type=tuple(out_type), mesh=mesh, scratch_types=scratch,
        compiler_params=pltpu.CompilerParams(use_tc_tiling_on_sc=False, needs_layout_passes=False),
    )
    return fn(z, dst2, zeros_nh, zeros_n)


def _node1_body(x_ref, wd_ref, wb_ref, p_ref, q_ref):
    x = x_ref[...]
    p_ref[...] = jnp.dot(x, wd_ref[...], preferred_element_type=jnp.float32)
    q_ref[...] = jnp.dot(x, wb_ref[...], preferred_element_type=jnp.float32)


def _node_body(sa_ref, sb_ref, cpa_ref, cpb_ref, gnw_ref, gnb_ref,
               gna_ref, wd_ref, wb_ref, p_ref, q_ref):
    sa = sa_ref[...]
    sb = sb_ref[...]
    dn = (((0,), (0,)), ((), ()))
    one32 = jnp.ones((cpa_ref.shape[0], 1), jnp.float32)
    cnt = (jax.lax.dot_general(cpa_ref[...], one32, dn,
                               preferred_element_type=jnp.float32)
           + jax.lax.dot_general(cpb_ref[...], one32, dn,
                                 preferred_element_type=jnp.float32))
    h = (sa[0] + sa[1] + sb[0] + sb[1]) / jnp.maximum(cnt, 1.0)
    mean = jnp.mean(h, axis=0, keepdims=True)
    out = h - gna_ref[...] * mean
    var = jnp.mean(out * out, axis=0, keepdims=True)
    h = gnw_ref[...] * out * jax.lax.rsqrt(var + 1e-5) + gnb_ref[...]
    p_ref[...] = jnp.dot(h, wd_ref[...], preferred_element_type=jnp.float32)
    q_ref[...] = jnp.dot(h, wb_ref[...], preferred_element_type=jnp.float32)


def _edge_mlp_body(pre_ref, b1_ref, w2_ref, b2_ref, w3_ref, b3_ref, z_ref):
    h = jnp.maximum(pre_ref[...] + b1_ref[...], 0.0)
    h = jnp.dot(h, w2_ref[...], preferred_element_type=jnp.float32) + b2_ref[...]
    h = jnp.maximum(h, 0.0)
    z_ref[...] = jnp.dot(h, w3_ref[...], preferred_element_type=jnp.float32) + b3_ref[...]


def _head_body(sa_ref, sb_ref, cpa_ref, cpb_ref, batch_ref, dw_ref,
               db_ref, ow_ref, ob_ref, out_ref):
    sa = sa_ref[...]
    sb = sb_ref[...]
    dn = (((0,), (0,)), ((), ()))
    one32 = jnp.ones((cpa_ref.shape[0], 1), jnp.float32)
    cnt = (jax.lax.dot_general(cpa_ref[...], one32, dn,
                               preferred_element_type=jnp.float32)
           + jax.lax.dot_general(cpb_ref[...], one32, dn,
                                 preferred_element_type=jnp.float32))
    h = (sa[0] + sa[1] + sb[0] + sb[1]) / jnp.maximum(cnt, 1.0)
    n, _ = h.shape
    g = out_ref.shape[0]
    gids = jax.lax.broadcasted_iota(jnp.int32, (n, g), 1)
    oh = (batch_ref[...] == gids).astype(jnp.float32)
    pooled_s = jax.lax.dot_general(oh, h, dn, preferred_element_type=jnp.float32)
    cnt_g = jax.lax.dot_general(oh, jnp.ones((n, 1), jnp.float32), dn,
                                preferred_element_type=jnp.float32)
    pooled = pooled_s / jnp.maximum(cnt_g, 1.0)
    d = jnp.maximum(jnp.dot(pooled, dw_ref[...],
                            preferred_element_type=jnp.float32) + db_ref[...], 0.0)
    logits = jnp.dot(d, ow_ref[...],
                     preferred_element_type=jnp.float32) + ob_ref[...]
    m = jnp.max(logits, axis=1, keepdims=True)
    e = jnp.exp(logits - m)
    out_ref[...] = e / jnp.sum(e, axis=1, keepdims=True)


def _node1(x, wd, wb):
    n = x.shape[0]
    h = wd.shape[1]
    return pl.pallas_call(
        _node1_body,
        out_shape=(jax.ShapeDtypeStruct((n, h), jnp.float32),
                   jax.ShapeDtypeStruct((n, h), jnp.float32)),
    )(x, wd, wb)


def _node(sa, sb, cpa, cpb, gnw, gnb, gna, wd, wb):
    n = sa.shape[1]
    h = wd.shape[1]
    return pl.pallas_call(
        _node_body,
        out_shape=(jax.ShapeDtypeStruct((n, h), jnp.float32),
                   jax.ShapeDtypeStruct((n, h), jnp.float32)),
    )(sa, sb, cpa, cpb, gnw, gnb, gna, wd, wb)


def _edge_mlp(pre, b1, w2, b2, w3, b3, block_rows=3200):
    e2, hh = pre.shape
    assert e2 % block_rows == 0
    grid = e2 // block_rows
    return pl.pallas_call(
        _edge_mlp_body,
        grid=(grid,),
        in_specs=[
            pl.BlockSpec((block_rows, hh), lambda i: (i, 0)),
            pl.BlockSpec((1, hh), lambda i: (0, 0)),
            pl.BlockSpec((hh, hh), lambda i: (0, 0)),
            pl.BlockSpec((1, hh), lambda i: (0, 0)),
            pl.BlockSpec((hh, hh), lambda i: (0, 0)),
            pl.BlockSpec((1, hh), lambda i: (0, 0)),
        ],
        out_specs=pl.BlockSpec((block_rows, hh), lambda i: (i, 0)),
        out_shape=jax.ShapeDtypeStruct((e2, hh), jnp.float32),
    )(pre, b1, w2, b2, w3, b3)


def _head(sa, sb, cpa, cpb, batch2d, dw, db, ow, ob, g):
    c = ow.shape[1]
    return pl.pallas_call(
        _head_body,
        out_shape=jax.ShapeDtypeStruct((g, c), jnp.float32),
    )(sa, sb, cpa, cpb, batch2d, dw, db, ow, ob)


def kernel(x, edge_index, batch,
           conv1_w1, conv1_b1, conv1_w2, conv1_b2, conv1_w3, conv1_b3,
           conv2_w1, conv2_b1, conv2_w2, conv2_b2, conv2_w3, conv2_b3,
           conv3_w1, conv3_b1, conv3_w2, conv3_b2, conv3_w3, conv3_b3,
           gn1_weight, gn1_bias, gn1_alpha, gn2_weight, gn2_bias, gn2_alpha,
           dense_w, dense_b, out_w, out_b):
    n, f_in = x.shape
    e = edge_index.shape[1]
    g = 128
    src2 = edge_index[0].reshape(e // _EB, _EB)
    dst2 = edge_index[1].reshape(e // _EB, _EB)

    def split_w(w):
        fi = w.shape[0] // 2
        return w[:fi] - w[fi:], w[fi:]

    w1d, w1b = split_w(conv1_w1)
    w2d, w2b = split_w(conv2_w1)
    w3d, w3b = split_w(conv3_w1)
    r = lambda v: v.reshape(1, -1)
    rp = lambda v: jnp.concatenate([v, v]).reshape(1, -1)

    def blk(w):
        hh = w.shape[0]
        z = jnp.zeros((2 * hh, 2 * hh), jnp.float32)
        return z.at[:hh, :hh].set(w).at[hh:, hh:].set(w)

    h = 64
    zeros_nh = jnp.zeros((n, h), jnp.float32)
    zeros_n = jnp.zeros((n,), jnp.float32)
    nb = e // _EB
    halves = [(dst2[:nb // 2], src2[:nb // 2]),
              (dst2[nb // 2:], src2[nb // 2:])]

    def layer(p, q, b1, w2, b2, w3, b3, with_counts):
        zs = []
        for dh, sh in halves:
            pre2 = _sc_gather(p, q, dh, sh).reshape(-1, 2 * h)
            z2 = _edge_mlp(pre2, rp(b1), blk(w2), rp(b2), blk(w3), rp(b3))
            zs.append(z2.reshape(-1, h))
        outs = []
        for (dh, _), z in zip(halves, zs):
            outs.append(_sc_scatter(z, dh, zeros_nh, zeros_n,
                                    with_counts=with_counts))
        return outs

    p, q = _node1(x, w1d, w1b)
    (sa, cpa), (sb, cpb) = layer(p, q, conv1_b1, conv1_w2, conv1_b2,
                                 conv1_w3, conv1_b3, True)

    p, q = _node(sa, sb, cpa, cpb, r(gn1_weight), r(gn1_bias), r(gn1_alpha),
                 w2d, w2b)
    (sa2,), (sb2,) = layer(p, q, conv2_b1, conv2_w2, conv2_b2, conv2_w3,
                           conv2_b3, False)

    p, q = _node(sa2, sb2, cpa, cpb, r(gn2_weight), r(gn2_bias),
                 r(gn2_alpha), w3d, w3b)
    (sa3,), (sb3,) = layer(p, q, conv3_b1, conv3_w2, conv3_b2, conv3_w3,
                           conv3_b3, False)

    return _head(sa3, sb3, cpa, cpb, batch.reshape(n, 1), dense_w,
                 r(dense_b), out_w, r(out_b), g)
